# baseline (speedup 1.0000x reference)
import jax
import jax.numpy as jnp
from jax import lax
from jax.experimental import pallas as pl
from jax.experimental.pallas import tpu as pltpu
from jax.experimental.pallas import tpu_sc as plsc

_D, _H, _W = 16, 64, 64
_HP, _WP = 66, 67
_NB = 512
_BLK = _D * _H * _W
_OUT_BLK = _HP * _WP
_OUT_PAD = 4432
_GUARD = 16
_NW = 32
_TC_NB = 416
_SC_NB = _NB - _TC_NB
_BPW = _SC_NB // _NW
_INV17 = float(1.0 / 17.0)


def _sc_body(x_hbm, out_hbm, in_v, out_v):
    wid = lax.axis_index("s") * 2 + lax.axis_index("c")
    lane = lax.iota(jnp.int32, 16)
    even_lane = (lane & 1) == 0
    is0 = lane == 0
    le1 = lane <= 1
    is15 = lane == 15
    ge14 = lane >= 14
    zero = jnp.zeros((16,), jnp.float32)

    def zbody(i, c):
        out_v[pl.ds(i * 16, 16)] = zero
        return c

    lax.fori_loop(0, _OUT_PAD // 16, zbody, 0)

    def gbody(g, c):
        sbid = wid * _BPW + g
        pltpu.sync_copy(x_hbm.at[pl.ds((_TC_NB + sbid) * _BLK, _BLK)],
                        in_v.at[pl.ds(_GUARD, _BLK)])

        def hbody(h, cc):
            hb = _GUARD + h * _W
            for w0 in (0, 16, 32, 48):
                acc = zero
                for d in range(_D):
                    base = hb + d * (_H * _W) + w0
                    xv = in_v[pl.ds(base, 16)]
                    l1 = in_v[pl.ds(base + 1, 16)]
                    l2 = in_v[pl.ds(base + 2, 16)]
                    r1 = in_v[pl.ds(base - 1, 16)]
                    r2 = in_v[pl.ds(base - 2, 16)]
                    if w0 == 48:
                        l1 = jnp.where(is15, 0.0, l1)
                        l2 = jnp.where(ge14, 0.0, l2)
                    if w0 == 0:
                        r1 = jnp.where(is0, 0.0, r1)
                        r2 = jnp.where(le1, 0.0, r2)
                    cge1 = xv >= l1
                    cge2 = xv >= l2
                    cgt1 = xv > r1
                    cgt2 = xv > r2
                    modd = cgt1 & cge1
                    mev = (cge1 & cge2) | (cgt1 & cgt2)
                    m = jnp.where(even_lane, modd, mev)
                    s = xv / (1.0 + jnp.abs(xv))
                    acc = acc + xv + jnp.where(m, s, zero)
                out_v[pl.ds((h + 1) * _WP + 1 + w0, 16)] = acc * _INV17
            return cc

        lax.fori_loop(0, _H, hbody, 0)
        pltpu.sync_copy(out_v, out_hbm.at[pl.ds(sbid * _OUT_PAD, _OUT_PAD)])
        return c

    lax.fori_loop(0, _BPW, gbody, 0)


def _tc_body(x_ref, out_ref):
    xb = x_ref[0]
    z1 = jnp.zeros((_D, _H, 1), jnp.float32)
    z2 = jnp.zeros((_D, _H, 2), jnp.float32)
    l1 = jnp.concatenate([xb[:, :, 1:], z1], axis=2)
    l2 = jnp.concatenate([xb[:, :, 2:], z2], axis=2)
    r1 = jnp.concatenate([z1, xb[:, :, :-1]], axis=2)
    r2 = jnp.concatenate([z2, xb[:, :, :-2]], axis=2)
    cge1 = xb >= l1
    cge2 = xb >= l2
    cgt1 = xb > r1
    cgt2 = xb > r2
    modd = cgt1 & cge1
    mev = (cge1 & cge2) | (cgt1 & cgt2)
    wpar = lax.broadcasted_iota(jnp.int32, (_D, _H, _W), 2) & 1
    s = xb / (1.0 + jnp.abs(xb))
    t = jnp.where(wpar == 0, jnp.where(modd, s, 0.0),
                  jnp.where(mev, s, 0.0))
    fused = xb + t
    res = jnp.sum(fused, axis=0) * _INV17
    out_ref[0] = jnp.zeros((_HP, _WP), jnp.float32)
    out_ref[0, 1:65, 1:65] = res


def _tc_call(x4):
    return pl.pallas_call(
        _tc_body,
        grid=(_TC_NB,),
        in_specs=[pl.BlockSpec((1, _D, _H, _W), lambda i: (i, 0, 0, 0))],
        out_specs=pl.BlockSpec((1, _HP, _WP), lambda i: (i, 0, 0)),
        out_shape=jax.ShapeDtypeStruct((_TC_NB, _HP, _WP), jnp.float32),
    )(x4)


def _sc_call(xf):
    run = pl.kernel(
        _sc_body,
        out_type=jax.ShapeDtypeStruct((_SC_NB * _OUT_PAD,), jnp.float32),
        mesh=plsc.VectorSubcoreMesh(core_axis_name="c", subcore_axis_name="s"),
        scratch_types=[
            pltpu.VMEM((_GUARD + _BLK + _GUARD,), jnp.float32),
            pltpu.VMEM((_OUT_PAD,), jnp.float32),
        ],
    )
    out = run(xf)
    return out.reshape(_SC_NB, _OUT_PAD)[:, :_OUT_BLK].reshape(
        _SC_NB, _HP, _WP)


@jax.jit
def kernel(x):
    n, ch, d, h, w = x.shape
    x4 = x.reshape(_NB, _D, _H, _W)
    sc_out = _sc_call(x.reshape(-1))
    tc_out = _tc_call(x4)
    out = jnp.concatenate([tc_out, sc_out], axis=0)
    return out.reshape(n, ch, _HP, _WP)


# speedup vs baseline: 1.1643x; 1.1643x over previous
import jax
import jax.numpy as jnp
from jax import lax
from jax.experimental import pallas as pl
from jax.experimental.pallas import tpu as pltpu
from jax.experimental.pallas import tpu_sc as plsc

_D, _H, _W = 16, 64, 64
_HP, _WP = 66, 67
_NB = 512
_BLK = _D * _H * _W
_OUT_BLK = _HP * _WP
_OUT_PAD = 4432
_GUARD = 16
_NW = 32
_BPW = _NB // _NW
_INV17 = float(1.0 / 17.0)


def _body(x_hbm, out_hbm, in_v, out_v):
    wid = lax.axis_index("s") * 2 + lax.axis_index("c")
    lane = lax.iota(jnp.int32, 16)
    even_lane = (lane & 1) == 0
    is0 = lane == 0
    le1 = lane <= 1
    is15 = lane == 15
    ge14 = lane >= 14
    zero = jnp.zeros((16,), jnp.float32)

    def zbody(i, c):
        out_v[pl.ds(i * 16, 16)] = zero
        return c

    lax.fori_loop(0, _OUT_PAD // 16, zbody, 0)

    def gbody(g, c):
        bid = wid * _BPW + g
        pltpu.sync_copy(x_hbm.at[pl.ds(bid * _BLK, _BLK)],
                        in_v.at[pl.ds(_GUARD, _BLK)])

        def hbody(h, cc):
            hb = _GUARD + h * _W
            for w0 in (0, 16, 32, 48):
                acc = zero
                for d in range(_D):
                    base = hb + d * (_H * _W) + w0
                    xv = in_v[pl.ds(base, 16)]
                    l1 = in_v[pl.ds(base + 1, 16)]
                    l2 = in_v[pl.ds(base + 2, 16)]
                    r1 = in_v[pl.ds(base - 1, 16)]
                    r2 = in_v[pl.ds(base - 2, 16)]
                    if w0 == 48:
                        l1 = jnp.where(is15, 0.0, l1)
                        l2 = jnp.where(ge14, 0.0, l2)
                    if w0 == 0:
                        r1 = jnp.where(is0, 0.0, r1)
                        r2 = jnp.where(le1, 0.0, r2)
                    modd = (xv > r1) & (xv >= l1)
                    mev = (xv >= jnp.maximum(l1, l2)) | \
                        (xv > jnp.maximum(r1, r2))
                    s = xv / (1.0 + jnp.abs(xv))
                    t = jnp.where(even_lane, jnp.where(modd, s, zero),
                                  jnp.where(mev, s, zero))
                    acc = acc + xv + t
                out_v[pl.ds((h + 1) * _WP + 1 + w0, 16)] = acc * _INV17
            return cc

        lax.fori_loop(0, _H, hbody, 0)
        pltpu.sync_copy(out_v, out_hbm.at[pl.ds(bid * _OUT_PAD, _OUT_PAD)])
        return c

    lax.fori_loop(0, _BPW, gbody, 0)


@jax.jit
def kernel(x):
    n, ch, d, h, w = x.shape
    xf = x.reshape(_NB * _BLK)
    run = pl.kernel(
        _body,
        out_type=jax.ShapeDtypeStruct((_NB * _OUT_PAD,), jnp.float32),
        mesh=plsc.VectorSubcoreMesh(core_axis_name="c", subcore_axis_name="s"),
        scratch_types=[
            pltpu.VMEM((_GUARD + _BLK + _GUARD,), jnp.float32),
            pltpu.VMEM((_OUT_PAD,), jnp.float32),
        ],
    )
    out = run(xf)
    return out.reshape(_NB, _OUT_PAD)[:, :_OUT_BLK].reshape(n, ch, _HP, _WP)
